# baseline (speedup 1.0000x reference)
import functools

import jax
import jax.numpy as jnp
from jax import lax
from jax.experimental import pallas as pl
from jax.experimental.pallas import tpu as pltpu
from jax.experimental.pallas import tpu_sc as plsc

_D = 64
_DP = 128
_NB = 4096 * 200
_NC, _NS = 2, 16
_NW = _NC * _NS
_BPW = _NB // _NW
_C = 128
_NCHUNK = _BPW // _C

_mesh = plsc.VectorSubcoreMesh(core_axis_name="c", subcore_axis_name="s")


@functools.partial(
    pl.kernel,
    out_type=jax.ShapeDtypeStruct((_NB, _D), jnp.float32),
    mesh=_mesh,
    scratch_types=[
        pltpu.VMEM((_BPW,), jnp.int32),
        pltpu.VMEM((_C, _DP), jnp.float32),
        pltpu.VMEM((_C, _D), jnp.float32),
        pltpu.SemaphoreType.DMA,
    ],
)
def _gather_kernel(idx_hbm, table_hbm, out_hbm, idx_v, rows128, rows64, sem):
    wid = lax.axis_index("s") * _NC + lax.axis_index("c")
    base = wid * _BPW
    pltpu.sync_copy(idx_hbm.at[pl.ds(base, _BPW)], idx_v)

    def body(c, carry):
        off = c * _C
        pltpu.async_copy(
            table_hbm.at[idx_v.at[pl.ds(off, _C)]], rows128, sem
        ).wait()
        def compact(r, carry2):
            for g in range(_D // 16):
                rows64[r, pl.ds(16 * g, 16)] = rows128[r, pl.ds(16 * g, 16)]
            return carry2

        lax.fori_loop(0, _C, compact, 0)
        pltpu.sync_copy(rows64, out_hbm.at[pl.ds(base + off, _C)])
        return carry

    lax.fori_loop(0, _NCHUNK, body, 0)


def kernel(x, table):
    idx = x.reshape(-1)
    tp = jnp.pad(table, ((0, 0), (0, _DP - _D)))
    out = _gather_kernel(idx, tp)
    return out.reshape(x.shape + (table.shape[1],))
